# baseline (speedup 1.0000x reference)
import functools

import jax
import jax.numpy as jnp
import numpy as np
from jax import lax
from jax.experimental import pallas as pl
from jax.experimental.pallas import tpu as pltpu
from jax.experimental.pallas import tpu_sc as plsc

_IMAX = 0x7FFFFFFF
_IMIN = -2147483648


def _stats_kernel(x_ref, o_ref):
    o_ref[...] = jnp.mean(x_ref[...], axis=1, keepdims=True)


def _mlp_kernel(stats_ref, w1_ref, b1_ref, w2_ref, b2_ref, keys_ref):
    cs = stats_ref[...][:, 0, :]
    h = jnp.dot(cs, w1_ref[...], preferred_element_type=jnp.float32)
    h = jnp.maximum(h + b1_ref[...], 0.0)
    s = jnp.dot(h, w2_ref[...], preferred_element_type=jnp.float32)
    s = s + b2_ref[...] + 0.0
    u = lax.bitcast_convert_type(s, jnp.int32)
    keys_ref[...] = jnp.where(u < 0, u ^ _IMAX, u)[:, None, :]


def _gather16(v, idx):
    return v.at[idx].get(mode="promise_in_bounds")


def _prefix_sum16(v):
    lane = lax.iota(jnp.int32, 16)
    for d in (1, 2, 4, 8):
        idx = jnp.maximum(lane - d, 0)
        sh = _gather16(v, idx)
        v = v + jnp.where(lane >= d, sh, 0).astype(jnp.int32)
    return v


def _lane_total16(v):
    lane = lax.iota(jnp.int32, 16)
    for d in (1, 2, 4, 8):
        v = v + _gather16(v, lane ^ d)
    return v


def _sc_mask_body(keys_hbm, mask_hbm, keys, mrow, *, rows, C, k, nc):
    wid = lax.axis_index("s") * nc + lax.axis_index("c")
    base = wid * rows
    pltpu.sync_copy(keys_hbm.at[pl.ds(base, rows)], keys)
    nv = C // 16
    for r in range(rows):
        def bit_body(i, t_u):
            sh = jnp.full((16,), 31, jnp.int32) - i
            cand = t_u | lax.shift_left(jnp.ones((16,), jnp.int32), sh)
            cand_s = cand ^ _IMIN
            cnt = jnp.zeros((16,), jnp.int32)
            for j in range(nv):
                kv = keys[r, pl.ds(j * 16, 16)]
                cnt = cnt + jnp.where(kv >= cand_s, 1, 0).astype(jnp.int32)
            total = _lane_total16(cnt)
            return jnp.where(total >= k, cand, t_u)

        t_u = lax.fori_loop(0, 32, bit_body, jnp.zeros((16,), jnp.int32))
        thr = t_u ^ _IMIN

        ngt = jnp.zeros((16,), jnp.int32)
        for j in range(nv):
            kv = keys[r, pl.ds(j * 16, 16)]
            ngt = ngt + jnp.where(kv > thr, 1, 0).astype(jnp.int32)
        budget = np.int32(k) - _lane_total16(ngt)

        carry = jnp.zeros((16,), jnp.int32)
        for j in range(nv):
            kv = keys[r, pl.ds(j * 16, 16)]
            eq = kv == thr
            eqi = jnp.where(eq, 1, 0).astype(jnp.int32)
            csum = _prefix_sum16(eqi) + carry
            sel = (kv > thr) | (eq & (csum <= budget))
            mrow[r, pl.ds(j * 16, 16)] = jnp.where(sel, 1.0, 0.0).astype(jnp.float32)
            carry = carry + _lane_total16(eqi)
    pltpu.sync_copy(mrow, mask_hbm.at[pl.ds(base, rows)])


def _apply_kernel(mask_ref, x_ref, out_ref, maske_ref):
    me = jnp.broadcast_to(mask_ref[...], out_ref.shape)
    out_ref[...] = x_ref[...] * me
    maske_ref[...] = me


def kernel(x, W1, b1, W2, b2, keep_ratio):
    B, N, C = x.shape
    k = max(1, C // 2)
    nc, ns = 2, 16
    rows = B // (nc * ns)

    stats = pl.pallas_call(
        _stats_kernel,
        grid=(B,),
        in_specs=[pl.BlockSpec((1, N, C), lambda b: (b, 0, 0))],
        out_specs=pl.BlockSpec((1, 1, C), lambda b: (b, 0, 0)),
        out_shape=jax.ShapeDtypeStruct((B, 1, C), jnp.float32),
    )(x)

    score_keys = pl.pallas_call(
        _mlp_kernel,
        out_shape=jax.ShapeDtypeStruct((B, 1, C), jnp.int32),
    )(stats, W1, b1.reshape(1, -1), W2, b2.reshape(1, -1))

    sc_mask = functools.partial(
        pl.kernel,
        mesh=plsc.VectorSubcoreMesh(core_axis_name="c", subcore_axis_name="s"),
        out_type=jax.ShapeDtypeStruct((B, C), jnp.float32),
        scratch_types=[
            pltpu.VMEM((rows, C), jnp.int32),
            pltpu.VMEM((rows, C), jnp.float32),
        ],
    )(functools.partial(_sc_mask_body, rows=rows, C=C, k=k, nc=nc))
    mask = sc_mask(score_keys.reshape(B, C))

    nsplit = 3
    nb = N // nsplit
    out, mask_e = pl.pallas_call(
        _apply_kernel,
        grid=(B, nsplit),
        in_specs=[
            pl.BlockSpec((1, 1, C), lambda b, n: (b, 0, 0)),
            pl.BlockSpec((1, nb, C), lambda b, n: (b, n, 0)),
        ],
        out_specs=[
            pl.BlockSpec((1, nb, C), lambda b, n: (b, n, 0)),
            pl.BlockSpec((1, nb, C), lambda b, n: (b, n, 0)),
        ],
        out_shape=[
            jax.ShapeDtypeStruct((B, N, C), jnp.float32),
            jax.ShapeDtypeStruct((B, N, C), jnp.float32),
        ],
    )(mask.reshape(B, 1, C), x)
    return (out, mask_e)


# speedup vs baseline: 1.3442x; 1.3442x over previous
import functools

import jax
import jax.numpy as jnp
import numpy as np
from jax import lax
from jax.experimental import pallas as pl
from jax.experimental.pallas import tpu as pltpu
from jax.experimental.pallas import tpu_sc as plsc

_IMAX = 0x7FFFFFFF
_IMIN = -2147483648


def _stats_kernel(x_ref, o_ref):
    o_ref[...] = jnp.mean(x_ref[...], axis=1, keepdims=True)


def _mlp_kernel(stats_ref, w1_ref, b1_ref, w2_ref, b2_ref, keys_ref):
    cs = stats_ref[...][:, 0, :]
    h = jnp.dot(cs, w1_ref[...], preferred_element_type=jnp.float32)
    h = jnp.maximum(h + b1_ref[...], 0.0)
    s = jnp.dot(h, w2_ref[...], preferred_element_type=jnp.float32)
    s = s + b2_ref[...] + 0.0
    u = lax.bitcast_convert_type(s, jnp.int32)
    keys_ref[...] = jnp.where(u < 0, u ^ _IMAX, u)[:, None, :]


def _gather16(v, idx):
    return v.at[idx].get(mode="promise_in_bounds")


def _prefix_sum16(v):
    lane = lax.iota(jnp.int32, 16)
    for d in (1, 2, 4, 8):
        idx = jnp.maximum(lane - d, 0)
        sh = _gather16(v, idx)
        v = v + jnp.where(lane >= d, sh, 0).astype(jnp.int32)
    return v


def _lane_total16(v):
    lane = lax.iota(jnp.int32, 16)
    for d in (1, 2, 4, 8):
        v = v + _gather16(v, lane ^ d)
    return v


def _sc_mask_body(keys_hbm, mask_hbm, keys, mrow, *, rows, C, k, nc):
    wid = lax.axis_index("s") * nc + lax.axis_index("c")
    base = wid * rows
    pltpu.sync_copy(keys_hbm.at[pl.ds(base, rows)], keys)
    nv = C // 16
    for r in range(rows):
        def bit_body(i, t_u):
            sh = jnp.full((16,), 31, jnp.int32) - i
            cand = t_u | lax.shift_left(jnp.ones((16,), jnp.int32), sh)
            cand_s = cand ^ _IMIN
            cnt = jnp.zeros((16,), jnp.int32)
            for j in range(nv):
                kv = keys[r, pl.ds(j * 16, 16)]
                cnt = cnt + jnp.where(kv >= cand_s, 1, 0).astype(jnp.int32)
            total = _lane_total16(cnt)
            return jnp.where(total >= k, cand, t_u)

        t_u = lax.fori_loop(0, 32, bit_body, jnp.zeros((16,), jnp.int32))
        thr = t_u ^ _IMIN

        ngt = jnp.zeros((16,), jnp.int32)
        for j in range(nv):
            kv = keys[r, pl.ds(j * 16, 16)]
            ngt = ngt + jnp.where(kv > thr, 1, 0).astype(jnp.int32)
        budget = np.int32(k) - _lane_total16(ngt)

        carry = jnp.zeros((16,), jnp.int32)
        for j in range(nv):
            kv = keys[r, pl.ds(j * 16, 16)]
            eq = kv == thr
            eqi = jnp.where(eq, 1, 0).astype(jnp.int32)
            csum = _prefix_sum16(eqi) + carry
            sel = (kv > thr) | (eq & (csum <= budget))
            mrow[r, pl.ds(j * 16, 16)] = jnp.where(sel, 1.0, 0.0).astype(jnp.float32)
            carry = carry + _lane_total16(eqi)
    pltpu.sync_copy(mrow, mask_hbm.at[pl.ds(base, rows)])


def _apply_kernel(mask_ref, x_ref, out_ref, maske_ref):
    me = jnp.broadcast_to(mask_ref[...], out_ref.shape)
    out_ref[...] = x_ref[...] * me
    maske_ref[...] = me


def kernel(x, W1, b1, W2, b2, keep_ratio):
    B, N, C = x.shape
    k = max(1, C // 2)
    nc, ns = 2, 16
    rows = B // (nc * ns)

    stats = pl.pallas_call(
        _stats_kernel,
        grid=(B,),
        in_specs=[pl.BlockSpec((1, N, C), lambda b: (b, 0, 0))],
        out_specs=pl.BlockSpec((1, 1, C), lambda b: (b, 0, 0)),
        out_shape=jax.ShapeDtypeStruct((B, 1, C), jnp.float32),
    )(x)

    score_keys = pl.pallas_call(
        _mlp_kernel,
        out_shape=jax.ShapeDtypeStruct((B, 1, C), jnp.int32),
    )(stats, W1, b1.reshape(1, -1), W2, b2.reshape(1, -1))

    sc_mask = functools.partial(
        pl.kernel,
        mesh=plsc.VectorSubcoreMesh(core_axis_name="c", subcore_axis_name="s"),
        out_type=jax.ShapeDtypeStruct((B, C), jnp.float32),
        scratch_types=[
            pltpu.VMEM((rows, C), jnp.int32),
            pltpu.VMEM((rows, C), jnp.float32),
        ],
    )(functools.partial(_sc_mask_body, rows=rows, C=C, k=k, nc=nc))
    mask = sc_mask(score_keys.reshape(B, C))

    bb = 2
    out, mask_e = pl.pallas_call(
        _apply_kernel,
        grid=(B // bb,),
        in_specs=[
            pl.BlockSpec((bb, 1, C), lambda b: (b, 0, 0)),
            pl.BlockSpec((bb, N, C), lambda b: (b, 0, 0)),
        ],
        out_specs=[
            pl.BlockSpec((bb, N, C), lambda b: (b, 0, 0)),
            pl.BlockSpec((bb, N, C), lambda b: (b, 0, 0)),
        ],
        out_shape=[
            jax.ShapeDtypeStruct((B, N, C), jnp.float32),
            jax.ShapeDtypeStruct((B, N, C), jnp.float32),
        ],
    )(mask.reshape(B, 1, C), x)
    return (out, mask_e)


# speedup vs baseline: 1.5134x; 1.1258x over previous
import functools

import jax
import jax.numpy as jnp
import numpy as np
from jax import lax
from jax.experimental import pallas as pl
from jax.experimental.pallas import tpu as pltpu
from jax.experimental.pallas import tpu_sc as plsc

_IMAX = 0x7FFFFFFF
_IMIN = -2147483648


def _stats_kernel(x_ref, o_ref):
    o_ref[...] = jnp.mean(x_ref[...], axis=1, keepdims=True)


def _mlp_kernel(stats_ref, w1_ref, b1_ref, w2_ref, b2_ref, keys_ref):
    cs = stats_ref[...][:, 0, :]
    h = jnp.dot(cs, w1_ref[...], preferred_element_type=jnp.float32)
    h = jnp.maximum(h + b1_ref[...], 0.0)
    s = jnp.dot(h, w2_ref[...], preferred_element_type=jnp.float32)
    s = s + b2_ref[...] + 0.0
    u = lax.bitcast_convert_type(s, jnp.int32)
    keys_ref[...] = jnp.where(u < 0, u ^ _IMAX, u)[:, None, :]


def _gather16(v, idx):
    return v.at[idx].get(mode="promise_in_bounds")


def _prefix_sum16(v):
    lane = lax.iota(jnp.int32, 16)
    for d in (1, 2, 4, 8):
        idx = jnp.maximum(lane - d, 0)
        sh = _gather16(v, idx)
        v = v + jnp.where(lane >= d, sh, 0).astype(jnp.int32)
    return v


def _lane_total16(v):
    lane = lax.iota(jnp.int32, 16)
    for d in (1, 2, 4, 8):
        v = v + _gather16(v, lane ^ d)
    return v


def _sc_mask_body(keys_hbm, mask_hbm, keys, mrow, *, rows, C, k, nc):
    wid = lax.axis_index("s") * nc + lax.axis_index("c")
    base = wid * rows
    pltpu.sync_copy(keys_hbm.at[pl.ds(base, rows)], keys)
    nv = C // 16
    for r in range(rows):
        def bit_body(i, t_u):
            sh = jnp.full((16,), 31, jnp.int32) - i
            cand = t_u | lax.shift_left(jnp.ones((16,), jnp.int32), sh)
            cand_s = cand ^ _IMIN
            cnt = jnp.zeros((16,), jnp.int32)
            for j in range(nv):
                kv = keys[r, pl.ds(j * 16, 16)]
                cnt = cnt + jnp.where(kv >= cand_s, 1, 0).astype(jnp.int32)
            total = _lane_total16(cnt)
            return jnp.where(total >= k, cand, t_u)

        t_u = lax.fori_loop(0, 32, bit_body, jnp.zeros((16,), jnp.int32))
        thr = t_u ^ _IMIN

        ngt = jnp.zeros((16,), jnp.int32)
        for j in range(nv):
            kv = keys[r, pl.ds(j * 16, 16)]
            ngt = ngt + jnp.where(kv > thr, 1, 0).astype(jnp.int32)
        budget = np.int32(k) - _lane_total16(ngt)

        carry = jnp.zeros((16,), jnp.int32)
        for j in range(nv):
            kv = keys[r, pl.ds(j * 16, 16)]
            eq = kv == thr
            eqi = jnp.where(eq, 1, 0).astype(jnp.int32)
            csum = _prefix_sum16(eqi) + carry
            sel = (kv > thr) | (eq & (csum <= budget))
            mrow[r, pl.ds(j * 16, 16)] = jnp.where(sel, 1.0, 0.0).astype(jnp.float32)
            carry = carry + _lane_total16(eqi)
    pltpu.sync_copy(mrow, mask_hbm.at[pl.ds(base, rows)])


def _apply_kernel(mask_ref, x_ref, out_ref, maske_ref):
    me = jnp.broadcast_to(mask_ref[...], out_ref.shape)
    out_ref[...] = x_ref[...] * me
    maske_ref[...] = me


def kernel(x, W1, b1, W2, b2, keep_ratio):
    B, N, C = x.shape
    k = max(1, C // 2)
    nc, ns = 2, 16
    rows = B // (nc * ns)

    sb = 2
    stats = pl.pallas_call(
        _stats_kernel,
        grid=(B // sb,),
        in_specs=[pl.BlockSpec((sb, N, C), lambda b: (b, 0, 0))],
        out_specs=pl.BlockSpec((sb, 1, C), lambda b: (b, 0, 0)),
        out_shape=jax.ShapeDtypeStruct((B, 1, C), jnp.float32),
    )(x)

    score_keys = pl.pallas_call(
        _mlp_kernel,
        out_shape=jax.ShapeDtypeStruct((B, 1, C), jnp.int32),
    )(stats, W1, b1.reshape(1, -1), W2, b2.reshape(1, -1))

    sc_mask = functools.partial(
        pl.kernel,
        mesh=plsc.VectorSubcoreMesh(core_axis_name="c", subcore_axis_name="s"),
        out_type=jax.ShapeDtypeStruct((B, C), jnp.float32),
        scratch_types=[
            pltpu.VMEM((rows, C), jnp.int32),
            pltpu.VMEM((rows, C), jnp.float32),
        ],
    )(functools.partial(_sc_mask_body, rows=rows, C=C, k=k, nc=nc))
    mask = sc_mask(score_keys.reshape(B, C))

    bb = 4
    out, mask_e = pl.pallas_call(
        _apply_kernel,
        grid=(B // bb,),
        in_specs=[
            pl.BlockSpec((bb, 1, C), lambda b: (b, 0, 0)),
            pl.BlockSpec((bb, N, C), lambda b: (b, 0, 0)),
        ],
        out_specs=[
            pl.BlockSpec((bb, N, C), lambda b: (b, 0, 0)),
            pl.BlockSpec((bb, N, C), lambda b: (b, 0, 0)),
        ],
        out_shape=[
            jax.ShapeDtypeStruct((B, N, C), jnp.float32),
            jax.ShapeDtypeStruct((B, N, C), jnp.float32),
        ],
    )(mask.reshape(B, 1, C), x)
    return (out, mask_e)


# speedup vs baseline: 1.5346x; 1.0140x over previous
import functools

import jax
import jax.numpy as jnp
import numpy as np
from jax import lax
from jax.experimental import pallas as pl
from jax.experimental.pallas import tpu as pltpu
from jax.experimental.pallas import tpu_sc as plsc

_IMAX = 0x7FFFFFFF
_IMIN = -2147483648


def _stats_kernel(x_ref, o_ref):
    o_ref[...] = jnp.mean(x_ref[...], axis=1, keepdims=True)


def _mlp_kernel(stats_ref, w1_ref, b1_ref, w2_ref, b2_ref, keys_ref):
    cs = stats_ref[...][:, 0, :]
    h = jnp.dot(cs, w1_ref[...], preferred_element_type=jnp.float32)
    h = jnp.maximum(h + b1_ref[...], 0.0)
    s = jnp.dot(h, w2_ref[...], preferred_element_type=jnp.float32)
    s = s + b2_ref[...] + 0.0
    u = lax.bitcast_convert_type(s, jnp.int32)
    keys_ref[...] = jnp.where(u < 0, u ^ _IMAX, u)[:, None, :]


def _gather16(v, idx):
    return v.at[idx].get(mode="promise_in_bounds")


def _prefix_sum16(v):
    lane = lax.iota(jnp.int32, 16)
    for d in (1, 2, 4, 8):
        idx = jnp.maximum(lane - d, 0)
        sh = _gather16(v, idx)
        v = v + jnp.where(lane >= d, sh, 0).astype(jnp.int32)
    return v


def _lane_total16(v):
    lane = lax.iota(jnp.int32, 16)
    for d in (1, 2, 4, 8):
        v = v + _gather16(v, lane ^ d)
    return v


def _sc_mask_body(keys_hbm, mask_hbm, keys, mrow, *, rows, C, k, nc):
    wid = lax.axis_index("s") * nc + lax.axis_index("c")
    base = wid * rows
    pltpu.sync_copy(keys_hbm.at[pl.ds(base, rows)], keys)
    nv = C // 16
    for r in range(rows):
        def bit_body(i, t_u):
            sh = jnp.full((16,), 31, jnp.int32) - i
            cand = t_u | lax.shift_left(jnp.ones((16,), jnp.int32), sh)
            cand_s = cand ^ _IMIN
            cnt = jnp.zeros((16,), jnp.int32)
            for j in range(nv):
                kv = keys[r, pl.ds(j * 16, 16)]
                cnt = cnt + jnp.where(kv >= cand_s, 1, 0).astype(jnp.int32)
            total = _lane_total16(cnt)
            return jnp.where(total >= k, cand, t_u)

        t_u = lax.fori_loop(0, 32, bit_body, jnp.zeros((16,), jnp.int32))
        thr = t_u ^ _IMIN

        ngt = jnp.zeros((16,), jnp.int32)
        for j in range(nv):
            kv = keys[r, pl.ds(j * 16, 16)]
            ngt = ngt + jnp.where(kv > thr, 1, 0).astype(jnp.int32)
        budget = np.int32(k) - _lane_total16(ngt)

        carry = jnp.zeros((16,), jnp.int32)
        for j in range(nv):
            kv = keys[r, pl.ds(j * 16, 16)]
            eq = kv == thr
            eqi = jnp.where(eq, 1, 0).astype(jnp.int32)
            csum = _prefix_sum16(eqi) + carry
            sel = (kv > thr) | (eq & (csum <= budget))
            mrow[r, pl.ds(j * 16, 16)] = jnp.where(sel, 1.0, 0.0).astype(jnp.float32)
            carry = carry + _lane_total16(eqi)
    pltpu.sync_copy(mrow, mask_hbm.at[pl.ds(base, rows)])


def _apply_kernel(mask_ref, x_ref, out_ref, maske_ref):
    me = jnp.broadcast_to(mask_ref[...], out_ref.shape)
    out_ref[...] = x_ref[...] * me
    maske_ref[...] = me


def kernel(x, W1, b1, W2, b2, keep_ratio):
    B, N, C = x.shape
    k = max(1, C // 2)
    nc, ns = 2, 16
    rows = B // (nc * ns)

    sb = 4
    stats = pl.pallas_call(
        _stats_kernel,
        grid=(B // sb,),
        in_specs=[pl.BlockSpec((sb, N, C), lambda b: (b, 0, 0))],
        out_specs=pl.BlockSpec((sb, 1, C), lambda b: (b, 0, 0)),
        out_shape=jax.ShapeDtypeStruct((B, 1, C), jnp.float32),
    )(x)

    score_keys = pl.pallas_call(
        _mlp_kernel,
        out_shape=jax.ShapeDtypeStruct((B, 1, C), jnp.int32),
    )(stats, W1, b1.reshape(1, -1), W2, b2.reshape(1, -1))

    sc_mask = functools.partial(
        pl.kernel,
        mesh=plsc.VectorSubcoreMesh(core_axis_name="c", subcore_axis_name="s"),
        out_type=jax.ShapeDtypeStruct((B, C), jnp.float32),
        scratch_types=[
            pltpu.VMEM((rows, C), jnp.int32),
            pltpu.VMEM((rows, C), jnp.float32),
        ],
    )(functools.partial(_sc_mask_body, rows=rows, C=C, k=k, nc=nc))
    mask = sc_mask(score_keys.reshape(B, C))

    bb = 4
    out, mask_e = pl.pallas_call(
        _apply_kernel,
        grid=(B // bb,),
        in_specs=[
            pl.BlockSpec((bb, 1, C), lambda b: (b, 0, 0)),
            pl.BlockSpec((bb, N, C), lambda b: (b, 0, 0)),
        ],
        out_specs=[
            pl.BlockSpec((bb, N, C), lambda b: (b, 0, 0)),
            pl.BlockSpec((bb, N, C), lambda b: (b, 0, 0)),
        ],
        out_shape=[
            jax.ShapeDtypeStruct((B, N, C), jnp.float32),
            jax.ShapeDtypeStruct((B, N, C), jnp.float32),
        ],
    )(mask.reshape(B, 1, C), x)
    return (out, mask_e)
